# baseline (speedup 1.0000x reference)
import functools

import numpy as np
import jax
import jax.numpy as jnp
from jax import lax
from jax.experimental import pallas as pl
from jax.experimental.pallas import tpu as pltpu
from jax.experimental.pallas import tpu_sc as plsc

_START_BETA = 0.0001
_END_BETA = 0.02
_TIMESTEPS = 1000
_B, _C, _H, _W = 512, 3, 128, 128
_INNER = _C * _H * _W

_NC, _NS = 2, 16
_NW = _NC * _NS
_BPW = _B // _NW


def _schedule_tables():
    betas = np.linspace(_START_BETA, _END_BETA, _TIMESTEPS).astype(np.float32)
    alphas = (1.0 - betas).astype(np.float32)
    ac = np.cumprod(alphas, dtype=np.float32)
    sqrt_ac = np.sqrt(ac).astype(np.float32)
    sqrt_omac = np.sqrt(1.0 - ac).astype(np.float32)
    return sqrt_ac, sqrt_omac


_TBL_A, _TBL_S = _schedule_tables()


def _gather_coeffs(t, tbl_a, tbl_s):
    mesh = plsc.VectorSubcoreMesh(core_axis_name="c", subcore_axis_name="s")

    @functools.partial(
        pl.kernel,
        out_type=(
            jax.ShapeDtypeStruct((_B,), jnp.float32),
            jax.ShapeDtypeStruct((_B,), jnp.float32),
        ),
        mesh=mesh,
        scratch_types=[
            pltpu.VMEM((_BPW,), jnp.int32),
            pltpu.VMEM((_BPW,), jnp.float32),
            pltpu.VMEM((_BPW,), jnp.float32),
            pltpu.SemaphoreType.DMA,
        ],
    )
    def gather_k(t_hbm, a_hbm, s_hbm, oa_hbm, os_hbm, idx_v, av, sv, sem):
        wid = lax.axis_index("s") * _NC + lax.axis_index("c")
        base = wid * _BPW
        pltpu.sync_copy(t_hbm.at[pl.ds(base, _BPW)], idx_v)
        pltpu.async_copy(a_hbm.at[idx_v], av, sem).wait()
        pltpu.async_copy(s_hbm.at[idx_v], sv, sem).wait()
        pltpu.sync_copy(av, oa_hbm.at[pl.ds(base, _BPW)])
        pltpu.sync_copy(sv, os_hbm.at[pl.ds(base, _BPW)])

    return gather_k(t, tbl_a, tbl_s)


def _apply_noise(x2, n2, a2, s2):
    bb, cw = 16, 6144
    grid = (_B // bb, _INNER // cw)

    def body(a_ref, s_ref, x_ref, n_ref, o_ref):
        o_ref[...] = a_ref[...] * x_ref[...] + s_ref[...] * n_ref[...]

    return pl.pallas_call(
        body,
        grid=grid,
        in_specs=[
            pl.BlockSpec((bb, 1), lambda i, j: (i, 0)),
            pl.BlockSpec((bb, 1), lambda i, j: (i, 0)),
            pl.BlockSpec((bb, cw), lambda i, j: (i, j)),
            pl.BlockSpec((bb, cw), lambda i, j: (i, j)),
        ],
        out_specs=pl.BlockSpec((bb, cw), lambda i, j: (i, j)),
        out_shape=jax.ShapeDtypeStruct((_B, _INNER), jnp.float32),
    )(a2, s2, x2, n2)


def kernel(original_images, noise, t):
    tbl_a = jnp.asarray(_TBL_A)
    tbl_s = jnp.asarray(_TBL_S)
    a, s = _gather_coeffs(t, tbl_a, tbl_s)
    x2 = original_images.reshape(_B, _INNER)
    n2 = noise.reshape(_B, _INNER)
    out = _apply_noise(x2, n2, a.reshape(_B, 1), s.reshape(_B, 1))
    return out.reshape(_B, _C, _H, _W)


# speedup vs baseline: 1.2731x; 1.2731x over previous
import numpy as np
import jax
import jax.numpy as jnp
from jax import lax
from jax.experimental import pallas as pl

_START_BETA = 0.0001
_END_BETA = 0.02
_TIMESTEPS = 1000
_B, _C, _H, _W = 512, 3, 128, 128
_INNER = _C * _H * _W
_TPAD = 1024


def _schedule_tables():
    betas = np.linspace(_START_BETA, _END_BETA, _TIMESTEPS).astype(np.float32)
    alphas = (1.0 - betas).astype(np.float32)
    ac = np.cumprod(alphas, dtype=np.float32)
    tbl = np.zeros((2, _TPAD), dtype=np.float32)
    tbl[0, :_TIMESTEPS] = np.sqrt(ac)
    tbl[1, :_TIMESTEPS] = np.sqrt(1.0 - ac)
    return tbl


_TBL = _schedule_tables()

_BB = 16


def _body(t_ref, tbl_ref, x_ref, n_ref, o_ref):
    tv = t_ref[...]
    k = lax.broadcasted_iota(jnp.int32, (_BB, _TPAD), 1)
    m = k == tv
    zero = jnp.zeros((), jnp.float32)
    a = jnp.sum(jnp.where(m, tbl_ref[0:1, :], zero), axis=1, keepdims=True)
    s = jnp.sum(jnp.where(m, tbl_ref[1:2, :], zero), axis=1, keepdims=True)
    o_ref[...] = a * x_ref[...] + s * n_ref[...]


def kernel(original_images, noise, t):
    tbl = jnp.asarray(_TBL)
    t2 = t.reshape(_B, 1)
    x2 = original_images.reshape(_B, _INNER)
    n2 = noise.reshape(_B, _INNER)
    grid = (_B // _BB,)
    out = pl.pallas_call(
        _body,
        grid=grid,
        in_specs=[
            pl.BlockSpec((_BB, 1), lambda i: (i, 0)),
            pl.BlockSpec((2, _TPAD), lambda i: (0, 0)),
            pl.BlockSpec((_BB, _INNER), lambda i: (i, 0)),
            pl.BlockSpec((_BB, _INNER), lambda i: (i, 0)),
        ],
        out_specs=pl.BlockSpec((_BB, _INNER), lambda i: (i, 0)),
        out_shape=jax.ShapeDtypeStruct((_B, _INNER), jnp.float32),
    )(t2, tbl, x2, n2)
    return out.reshape(_B, _C, _H, _W)


# speedup vs baseline: 1.2804x; 1.0057x over previous
import numpy as np
import jax
import jax.numpy as jnp
from jax import lax
from jax.experimental import pallas as pl
from jax.experimental.pallas import tpu as pltpu

_START_BETA = 0.0001
_END_BETA = 0.02
_TIMESTEPS = 1000
_B, _C, _H, _W = 512, 3, 128, 128
_INNER = _C * _H * _W
_TPAD = 1024

_BBC = 8
_NBUF = 8
_NCHUNK = _B // _BBC
_NOUTER = _NCHUNK // _NBUF


def _schedule_tables():
    betas = np.linspace(_START_BETA, _END_BETA, _TIMESTEPS).astype(np.float32)
    alphas = (1.0 - betas).astype(np.float32)
    ac = np.cumprod(alphas, dtype=np.float32)
    tbl = np.zeros((2, _TPAD), dtype=np.float32)
    tbl[0, :_TIMESTEPS] = np.sqrt(ac)
    tbl[1, :_TIMESTEPS] = np.sqrt(1.0 - ac)
    return tbl


_TBL = _schedule_tables()


def _body(t_ref, tbl_ref, x_hbm, n_hbm, o_hbm, xb, nb, ob, xsem, nsem, osem):
    i = pl.program_id(0)

    def rows(c):
        return pl.ds(pl.multiple_of(c * _BBC, _BBC), _BBC)

    def in_copies(c, b):
        cx = pltpu.make_async_copy(x_hbm.at[rows(c), :], xb.at[b], xsem.at[b])
        cn = pltpu.make_async_copy(n_hbm.at[rows(c), :], nb.at[b], nsem.at[b])
        return cx, cn

    def out_copy(c, b):
        return pltpu.make_async_copy(ob.at[b], o_hbm.at[rows(c), :], osem.at[b])

    @pl.when(i == 0)
    def _prologue():
        for b in range(_NBUF):
            cx, cn = in_copies(b, b)
            cx.start()
            cn.start()

    for b in range(_NBUF):
        c = i * _NBUF + b

        @pl.when(i > 0)
        def _free_out_slot(b=b):
            out_copy((i - 1) * _NBUF + b, b).wait()

        cx, cn = in_copies(c, b)
        cx.wait()
        cn.wait()

        tv = t_ref[rows(c), :]
        k = lax.broadcasted_iota(jnp.int32, (_BBC, _TPAD), 1)
        m = k == tv
        zero = jnp.zeros((), jnp.float32)
        a = jnp.sum(jnp.where(m, tbl_ref[0:1, :], zero), axis=1, keepdims=True)
        s = jnp.sum(jnp.where(m, tbl_ref[1:2, :], zero), axis=1, keepdims=True)
        ob[b, :, :] = a * xb[b, :, :] + s * nb[b, :, :]

        out_copy(c, b).start()

        @pl.when(c + _NBUF < _NCHUNK)
        def _issue_next_in(c=c, b=b):
            nx, nn = in_copies(c + _NBUF, b)
            nx.start()
            nn.start()

    @pl.when(i == _NOUTER - 1)
    def _epilogue():
        for b in range(_NBUF):
            out_copy((_NOUTER - 1) * _NBUF + b, b).wait()


def kernel(original_images, noise, t):
    tbl = jnp.asarray(_TBL)
    t2 = t.reshape(_B, 1)
    x2 = original_images.reshape(_B, _INNER)
    n2 = noise.reshape(_B, _INNER)
    out = pl.pallas_call(
        _body,
        grid=(_NOUTER,),
        in_specs=[
            pl.BlockSpec(memory_space=pltpu.VMEM),
            pl.BlockSpec(memory_space=pltpu.VMEM),
            pl.BlockSpec(memory_space=pl.ANY),
            pl.BlockSpec(memory_space=pl.ANY),
        ],
        out_specs=pl.BlockSpec(memory_space=pl.ANY),
        out_shape=jax.ShapeDtypeStruct((_B, _INNER), jnp.float32),
        scratch_shapes=[
            pltpu.VMEM((_NBUF, _BBC, _INNER), jnp.float32),
            pltpu.VMEM((_NBUF, _BBC, _INNER), jnp.float32),
            pltpu.VMEM((_NBUF, _BBC, _INNER), jnp.float32),
            pltpu.SemaphoreType.DMA((_NBUF,)),
            pltpu.SemaphoreType.DMA((_NBUF,)),
            pltpu.SemaphoreType.DMA((_NBUF,)),
        ],
    )(t2, tbl, x2, n2)
    return out.reshape(_B, _C, _H, _W)


# speedup vs baseline: 5.4126x; 4.2272x over previous
import numpy as np
import jax
import jax.numpy as jnp
from jax.experimental import pallas as pl
from jax.experimental.pallas import tpu as pltpu

_START_BETA = 0.0001
_END_BETA = 0.02
_TIMESTEPS = 1000
_B, _C, _H, _W = 512, 3, 128, 128

_BBC = 8
_NBUF = 8
_NCHUNK = _B // _BBC
_NOUTER = _NCHUNK // _NBUF


def _schedule_tables():
    betas = np.linspace(_START_BETA, _END_BETA, _TIMESTEPS).astype(np.float32)
    alphas = (1.0 - betas).astype(np.float32)
    ac = np.cumprod(alphas, dtype=np.float32)
    tbl = np.zeros((2, _TIMESTEPS), dtype=np.float32)
    tbl[0] = np.sqrt(ac)
    tbl[1] = np.sqrt(1.0 - ac)
    return tbl


_TBL = _schedule_tables()


def _body(t_ref, tbl_ref, x_hbm, n_hbm, o_hbm, xb, nb, ob, xsem, nsem, osem):
    i = pl.program_id(0)

    def rows(c):
        return pl.ds(pl.multiple_of(c * _BBC, _BBC), _BBC)

    def in_copies(c, b):
        cx = pltpu.make_async_copy(x_hbm.at[rows(c)], xb.at[b], xsem.at[b])
        cn = pltpu.make_async_copy(n_hbm.at[rows(c)], nb.at[b], nsem.at[b])
        return cx, cn

    def out_copy(c, b):
        return pltpu.make_async_copy(ob.at[b], o_hbm.at[rows(c)], osem.at[b])

    @pl.when(i == 0)
    def _prologue():
        for b in range(_NBUF):
            cx, cn = in_copies(b, b)
            cx.start()
            cn.start()

    for b in range(_NBUF):
        c = i * _NBUF + b

        @pl.when(i > 0)
        def _free_out_slot(b=b):
            out_copy((i - 1) * _NBUF + b, b).wait()

        cx, cn = in_copies(c, b)
        cx.wait()
        cn.wait()

        for r in range(_BBC):
            tv = t_ref[c * _BBC + r]
            a = tbl_ref[0, tv]
            s = tbl_ref[1, tv]
            ob[b, r] = a * xb[b, r] + s * nb[b, r]

        out_copy(c, b).start()

        @pl.when(c + _NBUF < _NCHUNK)
        def _issue_next_in(c=c, b=b):
            nx, nn = in_copies(c + _NBUF, b)
            nx.start()
            nn.start()

    @pl.when(i == _NOUTER - 1)
    def _epilogue():
        for b in range(_NBUF):
            out_copy((_NOUTER - 1) * _NBUF + b, b).wait()


def kernel(original_images, noise, t):
    tbl = jnp.asarray(_TBL)
    return pl.pallas_call(
        _body,
        grid=(_NOUTER,),
        in_specs=[
            pl.BlockSpec(memory_space=pltpu.SMEM),
            pl.BlockSpec(memory_space=pltpu.SMEM),
            pl.BlockSpec(memory_space=pl.ANY),
            pl.BlockSpec(memory_space=pl.ANY),
        ],
        out_specs=pl.BlockSpec(memory_space=pl.ANY),
        out_shape=jax.ShapeDtypeStruct((_B, _C, _H, _W), jnp.float32),
        scratch_shapes=[
            pltpu.VMEM((_NBUF, _BBC, _C, _H, _W), jnp.float32),
            pltpu.VMEM((_NBUF, _BBC, _C, _H, _W), jnp.float32),
            pltpu.VMEM((_NBUF, _BBC, _C, _H, _W), jnp.float32),
            pltpu.SemaphoreType.DMA((_NBUF,)),
            pltpu.SemaphoreType.DMA((_NBUF,)),
            pltpu.SemaphoreType.DMA((_NBUF,)),
        ],
    )(t, tbl, original_images, noise)
